# baseline (speedup 1.0000x reference)
import jax
import jax.numpy as jnp
from jax.experimental import pallas as pl
from jax.experimental.pallas import tpu as pltpu

_EPS2 = 1e-12
_BN = 1024
_BMC = 256


def _euclid_block(x_ref, w_ref, o_ref, xs_ref, x2_ref):
    @pl.when(pl.program_id(0) == 0)
    def _():
        xb = x_ref[...]
        x2 = jnp.sum(xb * xb, axis=1, keepdims=True)
        x2_ref[...] = jnp.broadcast_to(x2, x2_ref.shape)
        xs_ref[...] = (xb * -2.0).astype(jnp.bfloat16)
    wb = w_ref[...]
    wf = wb.astype(jnp.float32)
    w2 = jnp.sum(wf * wf, axis=0, keepdims=True)
    b = x_ref.shape[0]
    for i in range(0, b, _BMC):
        sl = pl.ds(i, _BMC)
        xw = jnp.dot(xs_ref[sl, :], wb, preferred_element_type=jnp.float32)
        d2 = jnp.maximum(xw + x2_ref[sl, :_BN] + w2, _EPS2)
        o_ref[sl, :] = d2 * jax.lax.rsqrt(d2)


def kernel(x, weight):
    b, k = x.shape
    _, o = weight.shape
    grid = (o // _BN,)
    return pl.pallas_call(
        _euclid_block,
        out_shape=jax.ShapeDtypeStruct((b, o), jnp.float32),
        grid=grid,
        in_specs=[
            pl.BlockSpec((b, k), lambda j: (0, 0)),
            pl.BlockSpec((k, _BN), lambda j: (0, j)),
        ],
        out_specs=pl.BlockSpec((b, _BN), lambda j: (0, j)),
        scratch_shapes=[
            pltpu.VMEM((b, k), jnp.bfloat16),
            pltpu.VMEM((b, _BN), jnp.float32),
        ],
        compiler_params=pltpu.CompilerParams(
            dimension_semantics=("arbitrary",),
            vmem_limit_bytes=58 * 1024 * 1024,
        ),
        name="euclidean_fused",
    )(x, weight.astype(jnp.bfloat16))


# speedup vs baseline: 1.4244x; 1.4244x over previous
import jax
import jax.numpy as jnp
from jax.experimental import pallas as pl
from jax.experimental.pallas import tpu as pltpu

_EPS2 = 1e-12
_BN = 1024
_BMC = 512


def _euclid_block(x_ref, w_ref, o_ref, xs_ref, x2_ref):
    @pl.when(pl.program_id(0) == 0)
    def _():
        xb = x_ref[...]
        x2 = jnp.sum(xb * xb, axis=1, keepdims=True)
        x2_ref[...] = jnp.broadcast_to(x2, x2_ref.shape)
        xs_ref[...] = xb * -2.0
    wb = w_ref[...]
    w2 = jnp.sum(wb * wb, axis=0, keepdims=True)
    b = x_ref.shape[0]
    for i in range(0, b, _BMC):
        sl = pl.ds(i, _BMC)
        xw = jnp.dot(xs_ref[sl, :], wb, preferred_element_type=jnp.float32)
        d2 = jnp.maximum(xw + x2_ref[sl, :_BN] + w2, _EPS2)
        o_ref[sl, :] = d2 * jax.lax.rsqrt(d2)


def kernel(x, weight):
    b, k = x.shape
    _, o = weight.shape
    grid = (o // _BN,)
    return pl.pallas_call(
        _euclid_block,
        out_shape=jax.ShapeDtypeStruct((b, o), jnp.float32),
        grid=grid,
        in_specs=[
            pl.BlockSpec((b, k), lambda j: (0, 0)),
            pl.BlockSpec((k, _BN), lambda j: (0, j)),
        ],
        out_specs=pl.BlockSpec((b, _BN), lambda j: (0, j)),
        scratch_shapes=[
            pltpu.VMEM((b, k), jnp.float32),
            pltpu.VMEM((b, _BN), jnp.float32),
        ],
        compiler_params=pltpu.CompilerParams(
            dimension_semantics=("arbitrary",),
            vmem_limit_bytes=58 * 1024 * 1024,
        ),
        name="euclidean_fused",
    )(x, weight)


# speedup vs baseline: 1.4768x; 1.0368x over previous
import jax
import jax.numpy as jnp
from jax.experimental import pallas as pl
from jax.experimental.pallas import tpu as pltpu

_EPS2 = 1e-12
_BN = 1024
_BMC = 512


def _euclid_block(x_ref, w_ref, o_ref, x2_ref):
    @pl.when(pl.program_id(0) == 0)
    def _():
        xb = x_ref[...]
        x2 = jnp.sum(xb * xb, axis=1, keepdims=True)
        x2_ref[...] = jnp.broadcast_to(x2, x2_ref.shape)
    wb = w_ref[...]
    w2 = jnp.sum(wb * wb, axis=0, keepdims=True)
    b = x_ref.shape[0]
    for i in range(0, b, _BMC):
        sl = pl.ds(i, _BMC)
        xw = jnp.dot(x_ref[sl, :], wb, preferred_element_type=jnp.float32)
        d2 = jnp.maximum(x2_ref[sl, :_BN] + w2 - 2.0 * xw, _EPS2)
        o_ref[sl, :] = d2 * jax.lax.rsqrt(d2)


def kernel(x, weight):
    b, k = x.shape
    _, o = weight.shape
    grid = (o // _BN,)
    return pl.pallas_call(
        _euclid_block,
        out_shape=jax.ShapeDtypeStruct((b, o), jnp.float32),
        grid=grid,
        in_specs=[
            pl.BlockSpec((b, k), lambda j: (0, 0)),
            pl.BlockSpec((k, _BN), lambda j: (0, j)),
        ],
        out_specs=pl.BlockSpec((b, _BN), lambda j: (0, j)),
        scratch_shapes=[
            pltpu.VMEM((b, _BN), jnp.float32),
        ],
        compiler_params=pltpu.CompilerParams(
            dimension_semantics=("arbitrary",),
            vmem_limit_bytes=58 * 1024 * 1024,
        ),
        name="euclidean_fused",
    )(x, weight)
